# baseline (speedup 1.0000x reference)
import functools

import jax
import jax.numpy as jnp
from jax import lax
from jax.experimental import pallas as pl
from jax.experimental.pallas import tpu as pltpu, tpu_sc as plsc

B = 64
INPUT_SIZE = 1024
MEM_SIZE = 16384
CELL_SIZE = 64
K = 8
VISIBLE = K + 1
NW = 32
BPW = B // NW
CHUNK = 128
NCHUNK = MEM_SIZE // CHUNK
L = 16

_mesh = plsc.VectorSubcoreMesh(core_axis_name="c", subcore_axis_name="s")

_F32_NEG_INF = float("-inf")


def _bf16r(v):
    u = lax.bitcast_convert_type(v, jnp.int32)
    lsb = lax.shift_right_logical(u, jnp.full(u.shape, 16, jnp.int32)) & 1
    r = (u + 0x7FFF + lsb) & jnp.int32(-65536)
    return lax.bitcast_convert_type(r, jnp.float32)


@functools.partial(
    pl.kernel,
    out_type=(
        jax.ShapeDtypeStruct((B, CELL_SIZE), jnp.float32),
        jax.ShapeDtypeStruct((B, L), jnp.int32),
        jax.ShapeDtypeStruct((B, L), jnp.float32),
    ),
    mesh=_mesh,
    scratch_types=[
        pltpu.VMEM((CHUNK, CELL_SIZE), jnp.float32),
        pltpu.VMEM((CHUNK, CELL_SIZE), jnp.float32),
        pltpu.VMEM((MEM_SIZE,), jnp.float32),
        pltpu.VMEM((L, INPUT_SIZE), jnp.float32),
        pltpu.VMEM((1, INPUT_SIZE), jnp.float32),
        pltpu.VMEM((1, CELL_SIZE), jnp.float32),
        pltpu.VMEM((1, L), jnp.int32),
        pltpu.VMEM((L, CELL_SIZE), jnp.float32),
        pltpu.VMEM((1, L), jnp.float32),
        pltpu.VMEM((1, CELL_SIZE), jnp.float32),
        pltpu.VMEM((1, L), jnp.int32),
        pltpu.SemaphoreType.DMA,
        pltpu.SemaphoreType.DMA,
        pltpu.SemaphoreType.DMA,
    ],
    compiler_params=pltpu.CompilerParams(needs_layout_passes=False),
)
def _sc_kernel(x_hbm, mem_hbm, lu_hbm, wq_hbm, bq_hbm,
               rv_out, pos_out, w_out,
               buf0, buf1, sims, wqb, xb, bqb, lub, rows, wnb, rvb, posb,
               sem0, sem1, semg):
    wid = lax.axis_index("s") * 2 + lax.axis_index("c")
    lane = lax.broadcasted_iota(jnp.int32, (L,), 0)
    lane0 = lane == 0

    pltpu.sync_copy(bq_hbm.at[pl.ds(0, 1)], bqb)

    for bi in range(BPW):
        b = wid * BPW + bi

        pltpu.sync_copy(x_hbm.at[pl.ds(b, 1)], xb)
        qv = []
        for t in range(CELL_SIZE // L):
            pltpu.sync_copy(wq_hbm.at[pl.ds(t * L, L)], wqb)

            def qbody(k16, acc):
                xv = _bf16r(xb[0, pl.ds(k16 * L, L)])
                for u in range(L):
                    kk = k16 * L + u
                    col = _bf16r(plsc.load_gather(
                        wqb, [lane, jnp.full((L,), kk, jnp.int32)]))
                    acc = acc + col * xv[u]
                return acc

            acc = lax.fori_loop(0, INPUT_SIZE // L, qbody,
                                jnp.zeros((L,), jnp.float32))
            qv.append(acc + bqb[0, pl.ds(t * L, L)])
        qs = [_bf16r(v) for v in qv]

        def compute_chunk(buf, off):
            def gbody(g, _):
                svec = jnp.full((L,), 0.0, jnp.float32)
                for rr in range(L):
                    r = g * L + rr
                    p = _bf16r(buf[r, pl.ds(0, L)]) * qs[0]
                    p = p + _bf16r(buf[r, pl.ds(L, L)]) * qs[1]
                    p = p + _bf16r(buf[r, pl.ds(2 * L, L)]) * qs[2]
                    p = p + _bf16r(buf[r, pl.ds(3 * L, L)]) * qs[3]
                    s = jnp.sum(p)
                    svec = jnp.where(lane == rr, s, svec)
                sims[pl.ds(off + g * L, L)] = svec
                return 0

            lax.fori_loop(0, CHUNK // L, gbody, 0)

        cp0 = pltpu.async_copy(mem_hbm.at[b, pl.ds(0, CHUNK)], buf0, sem0)

        def chunk_pair(j, _):
            off0 = (2 * j) * CHUNK
            off1 = (2 * j + 1) * CHUNK
            pltpu.make_async_copy(
                mem_hbm.at[b, pl.ds(0, CHUNK)], buf0, sem0).wait()
            pltpu.async_copy(mem_hbm.at[b, pl.ds(off1, CHUNK)], buf1, sem1)
            compute_chunk(buf0, off0)

            pltpu.make_async_copy(
                mem_hbm.at[b, pl.ds(0, CHUNK)], buf1, sem1).wait()

            @pl.when(j < NCHUNK // 2 - 1)
            def _():
                pltpu.async_copy(
                    mem_hbm.at[b, pl.ds(off1 + CHUNK, CHUNK)], buf0, sem0)

            compute_chunk(buf1, off1)
            return 0

        lax.fori_loop(0, NCHUNK // 2, chunk_pair, 0)

        pltpu.sync_copy(lu_hbm.at[pl.ds(b, 1)], lub)
        lu = lub[0, pl.ds(0, L)][0]
        picks = []
        pos_vec = jnp.full((L,), lu, jnp.int32)
        big = jnp.int32(2**30)
        for k in range(K):
            def sweep(i, carry):
                best, besti = carry
                for u in range(8):
                    base = i * 128 + u * L
                    v = sims[pl.ds(base, L)]
                    g = lane + base
                    m = v > best
                    best = jnp.where(m, v, best)
                    besti = jnp.where(m, g, besti)
                return best, besti

            best, besti = lax.fori_loop(
                0, MEM_SIZE // 128, sweep,
                (jnp.full((L,), _F32_NEG_INF, jnp.float32),
                 jnp.zeros((L,), jnp.int32)))
            m = jnp.max(best)
            idx = jnp.min(jnp.where(best == m, besti, big))
            picks.append(idx)
            pos_vec = jnp.where(lane == k, idx, pos_vec)
            plsc.store_scatter(
                sims, [jnp.full((L,), idx, jnp.int32)],
                jnp.full((L,), _F32_NEG_INF, jnp.float32), mask=lane0)

        picks.append(lu)
        cps = [
            pltpu.async_copy(mem_hbm.at[b, pl.ds(p, 1)],
                             rows.at[pl.ds(j, 1)], semg)
            for j, p in enumerate(picks)
        ]
        for cp in cps:
            cp.wait()

        wvec = jnp.full((L,), _F32_NEG_INF, jnp.float32)
        for j in range(VISIBLE):
            p = _bf16r(rows[j, pl.ds(0, L)]) * qs[0]
            p = p + _bf16r(rows[j, pl.ds(L, L)]) * qs[1]
            p = p + _bf16r(rows[j, pl.ds(2 * L, L)]) * qs[2]
            p = p + _bf16r(rows[j, pl.ds(3 * L, L)]) * qs[3]
            wvec = jnp.where(lane == j, jnp.sum(p), wvec)
        mw = jnp.max(wvec)
        e = jnp.where(lane < VISIBLE, jnp.exp(wvec - mw), 0.0)
        sw = jnp.sum(e)
        wn = e / sw
        wnb[0, :] = wn

        wnr = _bf16r(wn)
        rv = [jnp.zeros((L,), jnp.float32) for _ in range(4)]
        for j in range(VISIBLE):
            wj = wnr[j]
            for t in range(4):
                rv[t] = rv[t] + _bf16r(rows[j, pl.ds(t * L, L)]) * wj
        for t in range(4):
            rvb[0, pl.ds(t * L, L)] = rv[t]
        posb[0, :] = pos_vec

        pltpu.sync_copy(rvb, rv_out.at[pl.ds(b, 1)])
        pltpu.sync_copy(posb, pos_out.at[pl.ds(b, 1)])
        pltpu.sync_copy(wnb, w_out.at[pl.ds(b, 1)])


@jax.jit
def kernel(x, memory, least_used_mem, Wq, bq):
    lu16 = jnp.tile(least_used_mem, (1, L))
    bq2 = bq.reshape(1, CELL_SIZE)
    rv, pos, wsum = _sc_kernel(x, memory, lu16, Wq, bq2)
    return (rv.reshape(B, 1, CELL_SIZE), pos[:, :VISIBLE], wsum[:, :VISIBLE])
